# Initial kernel scaffold; baseline (speedup 1.0000x reference)
import jax
import jax.numpy as jnp
from jax.experimental import pallas as pl


def kernel(x, edge_index, batch, W1, b1, L1w, L1b, W2, b2, L2w, L2b, W3, b3):
    raise NotImplementedError("write your pallas kernel here")


# speedup vs baseline: 1.4954x; 1.4954x over previous
import functools

import jax
import jax.numpy as jnp
from jax import lax
from jax.experimental import pallas as pl
from jax.experimental.pallas import tpu as pltpu
from jax.experimental.pallas import tpu_sc as plsc

N = 10000
NPAD = 10240
D = 128
E = 320000
EL = E + N
NC, NS, L = 2, 16, 16
NW = NC * NS
ECH = 80
GCH = 81
ELP = NW * GCH * 128
G = 64
K = 30
BR = 1024
NEG = -3.0e38
BIG = 2 ** 30
JUNK = N
ZROW = NPAD - 1


def _mesh():
    return plsc.VectorSubcoreMesh(core_axis_name="c", subcore_axis_name="s",
                                  num_cores=NC, num_subcores=NS)


_SC_PARAMS = pltpu.CompilerParams(needs_layout_passes=False)


@functools.cache
def _deg_kernel_build():
    return pl.kernel(
        _deg_body,
        out_type=jax.ShapeDtypeStruct((NW, NPAD), jnp.float32),
        mesh=_mesh(),
        compiler_params=_SC_PARAMS,
        scratch_types=[
            pltpu.VMEM((ECH, 128), jnp.int32),
            pltpu.VMEM((NPAD,), jnp.float32),
        ],
    )


def _deg_body(col_hbm, deg_hbm, col_v, deg_v):
    wid = lax.axis_index("s") * NC + lax.axis_index("c")
    pltpu.sync_copy(col_hbm.at[wid], col_v)

    def zero(i, _):
        deg_v[pl.ds(i * L, L)] = jnp.zeros((L,), jnp.float32)
        return 0
    lax.fori_loop(0, NPAD // L, zero, 0)

    ones = jnp.ones((L,), jnp.float32)

    def body(k, _):
        idx = col_v[k // 8, pl.ds((k % 8) * L, L)]
        plsc.addupdate_scatter(deg_v, [idx], ones)
        return 0
    lax.fori_loop(0, ECH * 8, body, 0)
    pltpu.sync_copy(deg_v, deg_hbm.at[wid])


@functools.cache
def _gather_kernel_build():
    return pl.kernel(
        _gather_body,
        out_type=jax.ShapeDtypeStruct((ELP, D), jnp.float32),
        mesh=_mesh(),
        compiler_params=_SC_PARAMS,
        scratch_types=[
            pltpu.VMEM((GCH, 128), jnp.int32),
            pltpu.VMEM((128, D), jnp.float32),
            pltpu.VMEM((128, D), jnp.float32),
            pltpu.SemaphoreType.DMA,
            pltpu.SemaphoreType.DMA,
        ],
    )


def _gather_body(hw_hbm, row_hbm, gm_hbm, row_v, buf0, buf1, sem0, sem1):
    wid = lax.axis_index("s") * NC + lax.axis_index("c")
    base = wid * (GCH * 128)
    pltpu.sync_copy(row_hbm.at[wid], row_v)

    pltpu.async_copy(hw_hbm.at[row_v.at[0]], buf0, sem0).wait()

    def pair(p, _):
        j0 = 2 * p
        g1 = pltpu.async_copy(hw_hbm.at[row_v.at[j0 + 1]], buf1, sem1)
        pltpu.sync_copy(buf0, gm_hbm.at[pl.ds(base + j0 * 128, 128)])
        g1.wait()
        g2 = pltpu.async_copy(hw_hbm.at[row_v.at[j0 + 2]], buf0, sem0)
        pltpu.sync_copy(buf1, gm_hbm.at[pl.ds(base + (j0 + 1) * 128, 128)])
        g2.wait()
        return 0
    lax.fori_loop(0, (GCH - 1) // 2, pair, 0)
    pltpu.sync_copy(buf0, gm_hbm.at[pl.ds(base + (GCH - 1) * 128, 128)])


SPT = NPAD // NW


@functools.cache
def _keys_kernel_build():
    return pl.kernel(
        _keys_body,
        out_type=jax.ShapeDtypeStruct((NPAD,), jnp.float32),
        mesh=_mesh(),
        compiler_params=_SC_PARAMS,
        scratch_types=[
            pltpu.VMEM((SPT, D), jnp.float32),
            pltpu.VMEM((SPT,), jnp.float32),
        ],
    )


def _keys_body(h3_hbm, keys_hbm, rows_v, keys_v):
    wid = lax.axis_index("s") * NC + lax.axis_index("c")
    pltpu.sync_copy(h3_hbm.at[pl.ds(wid * SPT, SPT)], rows_v)
    c127 = jnp.full((L,), D - 1, jnp.int32)

    def body(i, _):
        r = lax.iota(jnp.int32, L) + i * L
        keys_v[pl.ds(i * L, L)] = plsc.load_gather(rows_v, [r, c127])
        return 0
    lax.fori_loop(0, SPT // L, body, 0)
    pltpu.sync_copy(keys_v, keys_hbm.at[pl.ds(wid * SPT, SPT)])


@functools.cache
def _pool_kernel_build():
    return pl.kernel(
        _pool_body,
        out_type=jax.ShapeDtypeStruct((G, K, D), jnp.float32),
        mesh=_mesh(),
        compiler_params=_SC_PARAMS,
        scratch_types=[
            pltpu.VMEM((NPAD,), jnp.float32),
            pltpu.VMEM((NPAD,), jnp.int32),
            pltpu.VMEM((NPAD,), jnp.float32),
            pltpu.VMEM((32,), jnp.int32),
            pltpu.VMEM((32, D), jnp.float32),
            pltpu.SemaphoreType.DMA,
        ],
    )


def _pool_body(h3_hbm, batch_hbm, keys_hbm, out_hbm,
               keys_v, batch_v, mk_v, idx_v, gbuf, sem):
    wid = lax.axis_index("s") * NC + lax.axis_index("c")
    pltpu.sync_copy(keys_hbm, keys_v)
    pltpu.sync_copy(batch_hbm, batch_v)
    lanes = lax.iota(jnp.int32, L)
    neg = jnp.float32(NEG)

    for u in range(2):
        g = wid * 2 + u

        def cnt(i, carry):
            c1, c2 = carry
            bv = batch_v[pl.ds(i * L, L)]
            c1 = c1 + plsc.all_reduce_population_count(bv < g)
            c2 = c2 + plsc.all_reduce_population_count(bv < g + 1)
            return (c1, c2)
        z = jnp.zeros((L,), jnp.int32)
        c1, c2 = lax.fori_loop(0, NPAD // L, cnt, (z, z))
        start = jnp.max(c1)
        end = jnp.max(c2)
        j0 = start // L
        j1 = (end + L - 1) // L

        def mask_body(j, _):
            gi = lanes + j * L
            kv = keys_v[pl.ds(j * L, L)]
            mk_v[pl.ds(j * L, L)] = jnp.where((gi >= start) & (gi < end), kv, neg)
            return 0
        lax.fori_loop(j0, j1, mask_body, 0)

        idx_v[pl.ds(0, L)] = jnp.full((L,), ZROW, jnp.int32)
        idx_v[pl.ds(L, L)] = jnp.full((L,), ZROW, jnp.int32)
        for p in range(K):
            def mx(j, m):
                return jnp.maximum(m, mk_v[pl.ds(j * L, L)])
            mvec = lax.fori_loop(j0, j1, mx, jnp.full((L,), neg, jnp.float32))
            mval = jnp.max(mvec)

            def am(j, b):
                v = mk_v[pl.ds(j * L, L)]
                gi = lanes + j * L
                return jnp.minimum(b, jnp.where(v == mval, gi, BIG))
            bvec = lax.fori_loop(j0, j1, am, jnp.full((L,), BIG, jnp.int32))
            w = jnp.min(bvec)
            valid = mval > jnp.float32(-1.0e37)
            sel = jnp.where(valid, w, jnp.int32(ZROW))
            plsc.store_scatter(idx_v, [jnp.full((L,), p, jnp.int32)],
                               jnp.full((L,), sel, jnp.int32),
                               mask=lanes == 0)

            @pl.when(valid)
            def _():
                plsc.store_scatter(mk_v, [jnp.full((L,), w, jnp.int32)],
                                   jnp.full((L,), neg, jnp.float32),
                                   mask=lanes == 0)

        pltpu.async_copy(h3_hbm.at[idx_v], gbuf, sem).wait()
        pltpu.sync_copy(gbuf.at[pl.ds(0, K)], out_hbm.at[g])


def _mm_body(x_ref, w_ref, o_ref):
    o_ref[...] = jnp.dot(x_ref[...], w_ref[...], preferred_element_type=jnp.float32)


def _mid_body(acc_ref, b_ref, lw_ref, lb_ref, w_ref, out_ref):
    b = b_ref[...][0:1, :]
    lb = lb_ref[...][0:1, :]
    o = acc_ref[...] + b
    t = jnp.maximum(o, 0.01 * o) + (jnp.dot(o, lw_ref[...],
                                            preferred_element_type=jnp.float32) + lb)
    out_ref[...] = jnp.dot(t, w_ref[...], preferred_element_type=jnp.float32)


def _fin_body(acc_ref, b_ref, out_ref):
    i = pl.program_id(0)
    b = b_ref[...][0:1, :]
    o = acc_ref[...] + b
    rid = lax.broadcasted_iota(jnp.int32, (BR, 1), 0) + i * BR
    out_ref[...] = jnp.where(rid < N, o, 0.0)


def _spec_rows(shape=(BR, D)):
    return pl.BlockSpec(shape, lambda i: (0,) * (len(shape) - 2) + (i, 0))


def _spec_const(shape):
    return pl.BlockSpec(shape, lambda i: (0,) * len(shape))


def _matmul(x, w):
    return pl.pallas_call(
        _mm_body,
        grid=(NPAD // BR,),
        in_specs=[_spec_rows(), _spec_const((D, D))],
        out_specs=_spec_rows(),
        out_shape=jax.ShapeDtypeStruct((NPAD, D), jnp.float32),
    )(x, w)


def _midstage(acc_pad, bb, lw, lb, w):
    return pl.pallas_call(
        _mid_body,
        grid=(NPAD // BR,),
        in_specs=[_spec_rows(), _spec_const((8, D)), _spec_const((D, D)),
                  _spec_const((8, D)), _spec_const((D, D))],
        out_specs=_spec_rows(),
        out_shape=jax.ShapeDtypeStruct((NPAD, D), jnp.float32),
    )(acc_pad, bb, lw, lb, w)


def kernel(x, edge_index, batch, W1, b1, L1w, L1b, W2, b2, L2w, L2b, W3, b3):
    f32, i32 = jnp.float32, jnp.int32
    row = edge_index[0]
    col = edge_index[1]
    loop = jnp.arange(N, dtype=i32)
    rowl = jnp.concatenate([row, loop])
    coll = jnp.concatenate([col, loop])
    rowlp = jnp.concatenate([rowl, jnp.zeros((ELP - EL,), i32)]).reshape(NW, GCH, 128)
    colp = jnp.concatenate([col, jnp.full((NW * ECH * 128 - E,), JUNK, i32)]
                           ).reshape(NW, ECH, 128)
    x_pad = jnp.concatenate([x, jnp.zeros((NPAD - N, D), f32)], axis=0)
    batchp = jnp.concatenate([batch, jnp.full((NPAD - N,), G, i32)])
    b1r = jnp.broadcast_to(b1[None, :], (8, D))
    b2r = jnp.broadcast_to(b2[None, :], (8, D))
    b3r = jnp.broadcast_to(b3[None, :], (8, D))
    l1br = jnp.broadcast_to(L1b[None, :], (8, D))
    l2br = jnp.broadcast_to(L2b[None, :], (8, D))

    degp = _deg_kernel_build()(colp)
    deg = jnp.sum(degp, axis=0)[:N] + 1.0
    dinv = jnp.where(deg > 0, lax.rsqrt(deg), 0.0)
    norm = dinv[rowl] * dinv[coll]

    gather = _gather_kernel_build()

    def conv_acc(hw):
        gm = gather(hw, rowlp)[:EL]
        msg = gm * norm[:, None]
        acc = jnp.zeros((N, D), f32).at[coll].add(msg)
        return jnp.concatenate([acc, jnp.zeros((NPAD - N, D), f32)], axis=0)

    hw1 = _matmul(x_pad, W1)
    acc1 = conv_acc(hw1)
    hw2 = _midstage(acc1, b1r, L1w, l1br, W2)
    acc2 = conv_acc(hw2)
    hw3 = _midstage(acc2, b2r, L2w, l2br, W3)
    acc3 = conv_acc(hw3)

    h3 = pl.pallas_call(
        _fin_body,
        grid=(NPAD // BR,),
        in_specs=[_spec_rows(), _spec_const((8, D))],
        out_specs=_spec_rows(),
        out_shape=jax.ShapeDtypeStruct((NPAD, D), f32),
    )(acc3, b3r)

    keys = _keys_kernel_build()(h3)
    pooled = _pool_kernel_build()(h3, batchp, keys)
    return pooled.reshape(G, K * D)
